# Initial kernel scaffold; baseline (speedup 1.0000x reference)
import jax
import jax.numpy as jnp
from jax.experimental import pallas as pl


def kernel(x, edge1, edge2, pos1, pos2, mask0, mask1, emb_table, gn_weight, gn_bias, gn_mean_scale, Wl1, bl1, Wr1, Wl2, bl2, Wr2, Wp, bp):
    raise NotImplementedError("write your pallas kernel here")


# speedup vs baseline: 2.9613x; 2.9613x over previous
import dataclasses
import functools

import jax
import jax.numpy as jnp
from jax import lax
from jax.experimental import pallas as pl
from jax.experimental.pallas import tpu as pltpu
from jax.experimental.pallas import tpu_sc as plsc

_MESH = dict(core_axis_name="c", subcore_axis_name="s")
C = 80
ZC = 40
F32 = jnp.float32
I32 = jnp.int32


def _vmesh():
    return plsc.VectorSubcoreMesh(**_MESH)


def _sc_cp(no_layout=False):
    cp = pltpu.CompilerParams()
    fields = pltpu.CompilerParams.__dataclass_fields__
    if "use_tc_tiling_on_sc" in fields:
        cp = dataclasses.replace(cp, use_tc_tiling_on_sc=False)
    if no_layout and "needs_layout_passes" in fields:
        cp = dataclasses.replace(cp, needs_layout_passes=False)
    return cp


def _fill(ref, n, value):
    @pl.loop(0, n)
    def _(i):
        ref.at[pl.ds(i, 1), pl.ds(0, 16)][...] = jnp.full((1, 16), value, F32)


def _sc_gather_rows(table, idx, D):
    B = idx.shape[0]
    nchunks = B // C
    nmax = (nchunks + 31) // 32

    @functools.partial(
        pl.kernel, mesh=_vmesh(),
        out_type=jax.ShapeDtypeStruct((B, D), F32),
        scratch_types=[pltpu.VMEM((C,), I32),
                       pltpu.VMEM((C, D), F32),
                       pltpu.SemaphoreType.DMA],
        compiler_params=_sc_cp())
    def k(table_hbm, idx_hbm, out_hbm, idx_v, rows_v, sem):
        c = lax.axis_index("c")
        s = lax.axis_index("s")
        w = s * 2 + c

        @pl.loop(0, nmax)
        def _(i):
            g = i * 32 + w

            @pl.when(g < nchunks)
            def _():
                base = g * C
                pltpu.sync_copy(idx_hbm.at[pl.ds(base, C)], idx_v)
                pltpu.async_copy(table_hbm.at[idx_v], rows_v, sem).wait()
                pltpu.sync_copy(rows_v, out_hbm.at[pl.ds(base, C)])

    return k(table, idx)


def _sc_segment_sum(tab_a, tab_b, src, dst, R):
    E = src.shape[0]
    nchunks = E // C
    nmax = (nchunks + 15) // 16
    nzero = R // C
    nzmax = (nzero + 15) // 16

    @functools.partial(
        pl.kernel, mesh=_vmesh(),
        out_type=(jax.ShapeDtypeStruct((R, 16), F32),
                  jax.ShapeDtypeStruct((R, 16), F32)),
        scratch_types=[pltpu.VMEM((C,), I32),
                       pltpu.VMEM((C,), I32),
                       pltpu.VMEM((C, 16), F32),
                       pltpu.VMEM((C, 16), F32),
                       pltpu.VMEM_SHARED((R, 16), F32),
                       pltpu.SemaphoreType.DMA],
        compiler_params=_sc_cp())
    def k(ta_hbm, tb_hbm, src_hbm, dst_hbm, oa_hbm, ob_hbm,
          sidx_v, didx_v, rows_v, zero_v, acc_sh, sem):
        c = lax.axis_index("c")
        s = lax.axis_index("s")
        _fill(zero_v, C, 0.0)

        @pl.loop(0, nzmax)
        def _(i):
            g = i * 16 + s

            @pl.when(g < nzero)
            def _():
                pltpu.sync_copy(zero_v, acc_sh.at[pl.ds(g * C, C)])

        plsc.subcore_barrier()

        @pl.loop(0, nmax)
        def _(i):
            g = i * 16 + s

            @pl.when(g < nchunks)
            def _():
                base = g * C
                pltpu.sync_copy(src_hbm.at[pl.ds(base, C)], sidx_v)
                pltpu.sync_copy(dst_hbm.at[pl.ds(base, C)], didx_v)

                @pl.when(c == 0)
                def _():
                    pltpu.async_copy(ta_hbm.at[sidx_v], rows_v, sem).wait()

                @pl.when(c == 1)
                def _():
                    pltpu.async_copy(tb_hbm.at[sidx_v], rows_v, sem).wait()

                pltpu.sync_copy(rows_v, acc_sh.at[didx_v], add=True)

        plsc.subcore_barrier()

        @pl.loop(0, nzmax)
        def _(i):
            g = i * 16 + s

            @pl.when(g < nzero)
            def _():
                @pl.when(c == 0)
                def _():
                    pltpu.sync_copy(acc_sh.at[pl.ds(g * C, C)],
                                    oa_hbm.at[pl.ds(g * C, C)])

                @pl.when(c == 1)
                def _():
                    pltpu.sync_copy(acc_sh.at[pl.ds(g * C, C)],
                                    ob_hbm.at[pl.ds(g * C, C)])

    return k(tab_a, tab_b, src, dst)


def _sc_segment_count(dst, R):
    E = dst.shape[0]
    Rh = R // 2
    nchunks = E // C
    nmax = (nchunks + 15) // 16
    nzero = Rh // ZC
    nzmax = (nzero + 15) // 16

    @functools.partial(
        pl.kernel, mesh=_vmesh(),
        out_type=jax.ShapeDtypeStruct((R, 16), F32),
        scratch_types=[pltpu.VMEM((C,), I32),
                       pltpu.VMEM((C,), I32),
                       pltpu.VMEM((C, 16), F32),
                       pltpu.VMEM((ZC, 16), F32),
                       pltpu.VMEM_SHARED((Rh + 8, 16), F32),
                       pltpu.SemaphoreType.DMA],
        compiler_params=_sc_cp())
    def k(dst_hbm, out_hbm, didx_v, cidx_v, ones_v, zero_v, acc_sh, sem):
        c = lax.axis_index("c")
        s = lax.axis_index("s")
        _fill(ones_v, C, 1.0)
        _fill(zero_v, ZC, 0.0)
        lo = c * Rh

        @pl.loop(0, nzmax)
        def _(i):
            g = i * 16 + s

            @pl.when(g < nzero)
            def _():
                pltpu.sync_copy(zero_v, acc_sh.at[pl.ds(g * ZC, ZC)])

        plsc.subcore_barrier()

        @pl.loop(0, nmax)
        def _(i):
            g = i * 16 + s

            @pl.when(g < nchunks)
            def _():
                pltpu.sync_copy(dst_hbm.at[pl.ds(g * C, C)], didx_v)
                for j in range(C // 16):
                    v = didx_v[pl.ds(j * 16, 16)]
                    local = v - lo
                    ok = (local >= 0) & (local < Rh)
                    cidx_v[pl.ds(j * 16, 16)] = jnp.where(ok, local, Rh)
                pltpu.sync_copy(ones_v, acc_sh.at[cidx_v], add=True)

        plsc.subcore_barrier()

        @pl.loop(0, nzmax)
        def _(i):
            g = i * 16 + s

            @pl.when(g < nzero)
            def _():
                pltpu.sync_copy(acc_sh.at[pl.ds(g * ZC, ZC)],
                                out_hbm.at[pl.ds(lo + g * ZC, ZC)])

    return k(dst)


def _sc_double_gather(pos2, mcat, h2):
    B = mcat.shape[0]
    M = pos2.shape[0]
    D = h2.shape[1]
    nchunks = B // C
    nmax = (nchunks + 31) // 32

    @functools.partial(
        pl.kernel, mesh=_vmesh(),
        out_type=jax.ShapeDtypeStruct((B, D), F32),
        scratch_types=[pltpu.VMEM((M,), I32),
                       pltpu.VMEM((C,), I32),
                       pltpu.VMEM((C,), I32),
                       pltpu.VMEM((C, D), F32),
                       pltpu.SemaphoreType.DMA],
        compiler_params=_sc_cp(no_layout=True))
    def k(pos2_hbm, mcat_hbm, h2_hbm, out_hbm, pos2_v, midx_v, gidx_v,
          rows_v, sem):
        c = lax.axis_index("c")
        s = lax.axis_index("s")
        w = s * 2 + c
        pltpu.sync_copy(pos2_hbm, pos2_v)

        @pl.loop(0, nmax)
        def _(i):
            g = i * 32 + w

            @pl.when(g < nchunks)
            def _():
                base = g * C
                pltpu.sync_copy(mcat_hbm.at[pl.ds(base, C)], midx_v)
                for j in range(C // 16):
                    mv = midx_v[pl.ds(j * 16, 16)]
                    gidx_v[pl.ds(j * 16, 16)] = plsc.load_gather(pos2_v, [mv])
                pltpu.async_copy(h2_hbm.at[gidx_v], rows_v, sem).wait()
                pltpu.sync_copy(rows_v, out_hbm.at[pl.ds(base, C)])

    return k(pos2, mcat, h2)


def _tc_stats(h0):
    N, D = h0.shape
    BN = 2000

    def body(h_ref, o_ref):
        i = pl.program_id(0)
        blk = h_ref[...]
        row = jnp.concatenate([jnp.sum(blk, axis=0)[None, :],
                               jnp.sum(blk * blk, axis=0)[None, :],
                               jnp.zeros((1, 128 - 2 * D), F32)], axis=1)
        st = jnp.concatenate([row, jnp.zeros((7, 128), F32)], axis=0)

        @pl.when(i == 0)
        def _():
            o_ref[...] = st

        @pl.when(i > 0)
        def _():
            o_ref[...] += st

    return pl.pallas_call(
        body, grid=(N // BN,),
        in_specs=[pl.BlockSpec((BN, D), lambda i: (i, 0))],
        out_specs=pl.BlockSpec((8, 128), lambda i: (0, 0)),
        out_shape=jax.ShapeDtypeStruct((8, 128), F32))(h0)


def _tc_norm_matmul(h0, stats, gw, gb, gms, Wl, bl, Wr):
    N, D = h0.shape
    BN = 2000
    fN = float(N)

    def body(h_ref, st_ref, gw_ref, gb_ref, gms_ref, wl_ref, bl_ref, wr_ref,
             za_ref, zb_ref, r_ref):
        mean = st_ref[0:1, 0:D] / fN
        msq = st_ref[0:1, D:2 * D] / fN
        cvec = mean * gms_ref[...]
        var = msq - 2.0 * cvec * mean + cvec * cvec
        a = gw_ref[...] * lax.rsqrt(var + 1e-5)
        hn = (h_ref[...] - cvec) * a + gb_ref[...]
        z = jnp.dot(hn, wl_ref[...], precision=lax.Precision.HIGHEST,
                    preferred_element_type=F32)
        za_ref[...] = z[:, 0:16]
        zb_ref[...] = z[:, 16:32]
        r_ref[...] = jnp.dot(hn, wr_ref[...], precision=lax.Precision.HIGHEST,
                             preferred_element_type=F32) + bl_ref[...]

    full = lambda *shape: pl.BlockSpec(shape, lambda i: tuple(0 for _ in shape))
    return pl.pallas_call(
        body, grid=(N // BN,),
        in_specs=[pl.BlockSpec((BN, D), lambda i: (i, 0)),
                  full(8, 128), full(1, D), full(1, D), full(1, D),
                  full(D, D), full(1, D), full(D, D)],
        out_specs=[pl.BlockSpec((BN, 16), lambda i: (i, 0)),
                   pl.BlockSpec((BN, 16), lambda i: (i, 0)),
                   pl.BlockSpec((BN, D), lambda i: (i, 0))],
        out_shape=[jax.ShapeDtypeStruct((N, 16), F32),
                   jax.ShapeDtypeStruct((N, 16), F32),
                   jax.ShapeDtypeStruct((N, D), F32)])(
                       h0, stats, gw, gb, gms, Wl, bl, Wr)


def _tc_matmul2(hp, Wl, bl, Wr):
    N2, DD = hp.shape
    BN = 2000

    def body(h_ref, wl_ref, bl_ref, wr_ref, za_ref, zb_ref, r_ref):
        hp_blk = h_ref[...]
        z = jnp.dot(hp_blk, wl_ref[...], precision=lax.Precision.HIGHEST,
                    preferred_element_type=F32)
        za_ref[...] = z[:, 0:16]
        zb_ref[...] = z[:, 16:32]
        r_ref[...] = jnp.dot(hp_blk, wr_ref[...], precision=lax.Precision.HIGHEST,
                             preferred_element_type=F32) + bl_ref[...]

    full = lambda *shape: pl.BlockSpec(shape, lambda i: tuple(0 for _ in shape))
    return pl.pallas_call(
        body, grid=(N2 // BN,),
        in_specs=[pl.BlockSpec((BN, DD), lambda i: (i, 0)),
                  full(DD, 32), full(1, 32), full(DD, 32)],
        out_specs=[pl.BlockSpec((BN, 16), lambda i: (i, 0)),
                   pl.BlockSpec((BN, 16), lambda i: (i, 0)),
                   pl.BlockSpec((BN, 32), lambda i: (i, 0))],
        out_shape=[jax.ShapeDtypeStruct((N2, 16), F32),
                   jax.ShapeDtypeStruct((N2, 16), F32),
                   jax.ShapeDtypeStruct((N2, 32), F32)])(hp, Wl, bl, Wr)


def _tc_combine(aa, ab, cnt, r):
    N, D = r.shape
    BN = 2000

    def body(aa_ref, ab_ref, c_ref, r_ref, o_ref):
        inv = 1.0 / jnp.maximum(c_ref[:, 0:1], 1.0)
        agg = jnp.concatenate([aa_ref[...], ab_ref[...]], axis=1) * inv
        o_ref[...] = jnp.maximum(agg + r_ref[...], 0.0)

    return pl.pallas_call(
        body, grid=(N // BN,),
        in_specs=[pl.BlockSpec((BN, 16), lambda i: (i, 0)),
                  pl.BlockSpec((BN, 16), lambda i: (i, 0)),
                  pl.BlockSpec((BN, 16), lambda i: (i, 0)),
                  pl.BlockSpec((BN, D), lambda i: (i, 0))],
        out_specs=pl.BlockSpec((BN, D), lambda i: (i, 0)),
        out_shape=jax.ShapeDtypeStruct((N, D), F32))(aa, ab, cnt, r)


def _tc_project(g0, g1, wp0, wp1, bp):
    P, D = g0.shape

    def body(g0_ref, g1_ref, w0_ref, w1_ref, bp_ref, o_ref):
        acc = g0_ref[...] * w0_ref[...] + g1_ref[...] * w1_ref[...]
        o_ref[...] = jnp.sum(acc, axis=1, keepdims=True) + bp_ref[...]

    full = lambda *shape: pl.BlockSpec(shape, lambda: tuple(0 for _ in shape))
    return pl.pallas_call(
        body,
        in_specs=[full(P, D), full(P, D), full(1, D), full(1, D), full(1, 1)],
        out_specs=full(P, 1),
        out_shape=jax.ShapeDtypeStruct((P, 1), F32))(g0, g1, wp0, wp1, bp)


def kernel(x, edge1, edge2, pos1, pos2, mask0, mask1, emb_table,
           gn_weight, gn_bias, gn_mean_scale,
           Wl1, bl1, Wr1, Wl2, bl2, Wr2, Wp, bp):
    N, D = emb_table.shape
    N2 = pos1.shape[0]
    M = pos2.shape[0]
    P = mask0.shape[0]

    x = x.astype(I32)
    src1 = edge1[0].astype(I32)
    dst1 = edge1[1].astype(I32)
    src2 = edge2[0].astype(I32)
    dst2 = edge2[1].astype(I32)
    pos1f = pos1.astype(I32).reshape(-1)
    pos2 = pos2.astype(I32)
    mcat = jnp.concatenate([mask0, mask1]).astype(I32)

    gw = gn_weight.reshape(1, D)
    gb = gn_bias.reshape(1, D)
    gms = gn_mean_scale.reshape(1, D)
    bl1r = bl1.reshape(1, D)
    bl2r = bl2.reshape(1, 32)
    wp0 = Wp[:D].reshape(1, D)
    wp1 = Wp[D:].reshape(1, D)
    bpr = bp.reshape(1, 1)

    h0 = _sc_gather_rows(emb_table, x, D)
    stats = _tc_stats(h0)
    z1a, z1b, r1 = _tc_norm_matmul(h0, stats, gw, gb, gms, Wl1, bl1r, Wr1)

    a1, b1 = _sc_segment_sum(z1a, z1b, src1, dst1, N)
    c1 = _sc_segment_count(dst1, N)
    h1 = _tc_combine(a1, b1, c1, r1)

    h1p = _sc_gather_rows(h1, pos1f, D).reshape(N2, 2 * D)
    z2a, z2b, r2 = _tc_matmul2(h1p, Wl2, bl2r, Wr2)
    a2, b2 = _sc_segment_sum(z2a, z2b, src2, dst2, N2)
    c2 = _sc_segment_count(dst2, N2)
    h2 = _tc_combine(a2, b2, c2, r2)

    gcat = _sc_double_gather(pos2, mcat, h2)
    return _tc_project(gcat[:P], gcat[P:], wp0, wp1, bpr)


# speedup vs baseline: 6.3712x; 2.1515x over previous
import dataclasses
import functools

import jax
import jax.numpy as jnp
from jax import lax
from jax.experimental import pallas as pl
from jax.experimental.pallas import tpu as pltpu
from jax.experimental.pallas import tpu_sc as plsc

_MESH = dict(core_axis_name="c", subcore_axis_name="s")
C = 80
ZC = 40
F32 = jnp.float32
I32 = jnp.int32


def _vmesh():
    return plsc.VectorSubcoreMesh(**_MESH)


def _sc_cp(no_layout=False):
    cp = pltpu.CompilerParams()
    fields = pltpu.CompilerParams.__dataclass_fields__
    if "use_tc_tiling_on_sc" in fields:
        cp = dataclasses.replace(cp, use_tc_tiling_on_sc=False)
    if no_layout and "needs_layout_passes" in fields:
        cp = dataclasses.replace(cp, needs_layout_passes=False)
    return cp


def _fill(ref, n, value):
    @pl.loop(0, n)
    def _(i):
        ref.at[pl.ds(i, 1), pl.ds(0, 16)][...] = jnp.full((1, 16), value, F32)


def _sc_gather_rows(table, idx, D):
    B = idx.shape[0]
    nchunks = B // C
    nmax = (nchunks + 31) // 32

    @functools.partial(
        pl.kernel, mesh=_vmesh(),
        out_type=jax.ShapeDtypeStruct((B, D), F32),
        scratch_types=[pltpu.VMEM((C,), I32),
                       pltpu.VMEM((C, D), F32),
                       pltpu.SemaphoreType.DMA],
        compiler_params=_sc_cp())
    def k(table_hbm, idx_hbm, out_hbm, idx_v, rows_v, sem):
        c = lax.axis_index("c")
        s = lax.axis_index("s")
        w = s * 2 + c

        @pl.loop(0, nmax)
        def _(i):
            g = i * 32 + w

            @pl.when(g < nchunks)
            def _():
                base = g * C
                pltpu.sync_copy(idx_hbm.at[pl.ds(base, C)], idx_v)
                pltpu.async_copy(table_hbm.at[idx_v], rows_v, sem).wait()
                pltpu.sync_copy(rows_v, out_hbm.at[pl.ds(base, C)])

    return k(table, idx)


_BK = 8
_ZB = 125


def _zero_acc(acc_sh, zero_v, R, s):
    nchunks = R // _ZB
    nmax = (nchunks + 15) // 16

    @pl.loop(0, nmax)
    def _(i):
        g = i * 16 + s

        @pl.when(g < nchunks)
        def _():
            pltpu.sync_copy(zero_v, acc_sh.at[pl.ds(g * _ZB, _ZB)])


def _clamp_block(didx_bj, cidx_bj, lo, Rh, R):
    for t in range(128 // 16):
        v = didx_bj[pl.ds(t * 16, 16)]
        local = v - lo
        ok = (local >= 0) & (local < Rh)
        cidx_bj[pl.ds(t * 16, 16)] = jnp.where(ok, local, Rh)


def _sc_segment_sum(tab_a, tab_b, src2d, dst2d, R, Rcnt=None, bk=_BK):
    nrows = src2d.shape[0]
    nblk = nrows // bk
    nmax = (nblk + 15) // 16
    fused = Rcnt is not None
    Rh = (Rcnt // 2) if fused else 0
    out_t = [jax.ShapeDtypeStruct((R, 16), F32),
             jax.ShapeDtypeStruct((R, 16), F32)]
    scratch = [pltpu.VMEM((2, bk, 128), I32),
               pltpu.VMEM((2, bk, 128), I32),
               pltpu.VMEM((2, bk, 128, 16), F32),
               pltpu.VMEM((_ZB, 16), F32),
               pltpu.VMEM_SHARED((R + 8, 16), F32),
               pltpu.SemaphoreType.DMA,
               pltpu.SemaphoreType.DMA,
               pltpu.SemaphoreType.DMA]
    if fused:
        out_t.append(jax.ShapeDtypeStruct((Rcnt, 16), F32))
        scratch += [pltpu.VMEM((2, bk, 128), I32),
                    pltpu.VMEM((128, 16), F32),
                    pltpu.VMEM_SHARED((Rh + 8, 16), F32)]

    @functools.partial(pl.kernel, mesh=_vmesh(), out_type=tuple(out_t),
                       scratch_types=scratch, compiler_params=_sc_cp())
    def k(ta_hbm, tb_hbm, src_hbm, dst_hbm, oa_hbm, ob_hbm, *rest):
        if fused:
            (oc_hbm, sidx_v, didx_v, rows_v, zero_v, acc_sh,
             semI, semG, semS, cidx_v, ones_v, accc_sh) = rest
        else:
            (sidx_v, didx_v, rows_v, zero_v, acc_sh,
             semI, semG, semS) = rest
        c = lax.axis_index("c")
        s = lax.axis_index("s")
        nmine = (nblk + 15 - s) // 16
        _fill(zero_v, _ZB, 0.0)
        if fused:
            _fill(ones_v, 128, 1.0)
            lo = c * Rh
        _zero_acc(acc_sh, zero_v, R, s)
        if fused:
            _zero_acc(accc_sh, zero_v, Rh, s)
        plsc.subcore_barrier()

        tab = [ta_hbm, tb_hbm]
        nscat = bk * (2 if fused else 1)

        def fire_idx(b, i):
            r0 = (i * 16 + s) * bk
            pltpu.async_copy(src_hbm.at[pl.ds(r0, bk)], sidx_v.at[b], semI)
            pltpu.async_copy(dst_hbm.at[pl.ds(r0, bk)], didx_v.at[b], semI)

        def wait_idx(b):
            for _ in range(2):
                pltpu.make_async_copy(src_hbm.at[pl.ds(0, bk)],
                                      sidx_v.at[b], semI).wait()

        def drain_scatters():
            for _ in range(nscat):
                pltpu.make_async_copy(rows_v.at[0, 0],
                                      acc_sh.at[didx_v.at[0, 0]], semS).wait()

        @pl.when(nmine > 0)
        def _():
            fire_idx(0, 0)

        for half in range(2):
            @pl.when(c == half)
            def _():
                @pl.loop(0, (nmax + 1) // 2)
                def _(o):
                    for b in range(2):
                        i = o * 2 + b

                        @pl.when(i < nmine)
                        def _():
                            @pl.when(i >= 1)
                            def _():
                                drain_scatters()

                            @pl.when(i + 1 < nmine)
                            def _():
                                fire_idx(1 - b, i + 1)

                            wait_idx(b)
                            for j in range(bk):
                                pltpu.async_copy(
                                    tab[half].at[sidx_v.at[b, j]],
                                    rows_v.at[b, j], semG)
                            for j in range(bk):
                                pltpu.make_async_copy(
                                    tab[half].at[sidx_v.at[b, 0]],
                                    rows_v.at[b, 0], semG).wait()
                            if fused:
                                for j in range(bk):
                                    _clamp_block(didx_v.at[b, j],
                                                 cidx_v.at[b, j], lo, Rh, R)
                            for j in range(bk):
                                pltpu.async_copy(
                                    rows_v.at[b, j],
                                    acc_sh.at[didx_v.at[b, j]], semS,
                                    add=True)
                            if fused:
                                for j in range(bk):
                                    pltpu.async_copy(
                                        ones_v, accc_sh.at[cidx_v.at[b, j]],
                                        semS, add=True)

        @pl.when(nmine >= 1)
        def _():
            drain_scatters()

        plsc.subcore_barrier()
        R16 = R // 16
        for half in range(2):
            @pl.when(c == half)
            def _():
                pltpu.sync_copy(acc_sh.at[pl.ds(s * R16, R16)],
                                [oa_hbm, ob_hbm][half].at[pl.ds(s * R16, R16)])
        if fused:
            ncv = Rh // _ZB
            ncmax = (ncv + 15) // 16

            @pl.loop(0, ncmax)
            def _(i):
                g = i * 16 + s

                @pl.when(g < ncv)
                def _():
                    pltpu.sync_copy(accc_sh.at[pl.ds(g * _ZB, _ZB)],
                                    oc_hbm.at[pl.ds(lo + g * _ZB, _ZB)])

    return k(tab_a, tab_b, src2d, dst2d)


def _sc_segment_count(dst2d, R):
    nrows = dst2d.shape[0]
    nblk = nrows // _BK
    nmax = (nblk + 15) // 16
    Rh = R // 2

    @functools.partial(
        pl.kernel, mesh=_vmesh(),
        out_type=jax.ShapeDtypeStruct((R, 16), F32),
        scratch_types=[pltpu.VMEM((2, _BK, 128), I32),
                       pltpu.VMEM((2, _BK, 128), I32),
                       pltpu.VMEM((128, 16), F32),
                       pltpu.VMEM((_ZB, 16), F32),
                       pltpu.VMEM_SHARED((Rh + 8, 16), F32),
                       pltpu.SemaphoreType.DMA,
                       pltpu.SemaphoreType.DMA],
        compiler_params=_sc_cp())
    def k(dst_hbm, out_hbm, didx_v, cidx_v, ones_v, zero_v, acc_sh,
          semI, semS):
        c = lax.axis_index("c")
        s = lax.axis_index("s")
        nmine = (nblk + 15 - s) // 16
        lo = c * Rh
        _fill(ones_v, 128, 1.0)
        _fill(zero_v, _ZB, 0.0)
        _zero_acc(acc_sh, zero_v, Rh, s)
        plsc.subcore_barrier()

        def fire_idx(b, i):
            r0 = (i * 16 + s) * _BK
            pltpu.async_copy(dst_hbm.at[pl.ds(r0, _BK)], didx_v.at[b], semI)

        @pl.when(nmine > 0)
        def _():
            fire_idx(0, 0)

        @pl.loop(0, (nmax + 1) // 2)
        def _(o):
            for b in range(2):
                i = o * 2 + b

                @pl.when(i < nmine)
                def _():
                    @pl.when(i >= 2)
                    def _():
                        for j in range(_BK):
                            pltpu.make_async_copy(
                                ones_v, acc_sh.at[cidx_v.at[0, 0]],
                                semS).wait()

                    @pl.when(i + 1 < nmine)
                    def _():
                        fire_idx(1 - b, i + 1)
                    pltpu.make_async_copy(dst_hbm.at[pl.ds(0, _BK)],
                                          didx_v.at[b], semI).wait()
                    for j in range(_BK):
                        _clamp_block(didx_v.at[b, j], cidx_v.at[b, j],
                                     lo, Rh, R)
                    for j in range(_BK):
                        pltpu.async_copy(ones_v, acc_sh.at[cidx_v.at[b, j]],
                                         semS, add=True)

        for thresh in (1, 2):
            @pl.when(nmine >= thresh)
            def _():
                for j in range(_BK):
                    pltpu.make_async_copy(ones_v, acc_sh.at[cidx_v.at[0, 0]],
                                          semS).wait()

        plsc.subcore_barrier()
        R16 = Rh // 16
        pltpu.sync_copy(acc_sh.at[pl.ds(s * R16, R16)],
                        out_hbm.at[pl.ds(lo + s * R16, R16)])

    return k(dst2d)


def _sc_double_gather(pos2, mcat, h2):
    B = mcat.shape[0]
    M = pos2.shape[0]
    D = h2.shape[1]
    nchunks = B // C
    nmax = (nchunks + 31) // 32

    @functools.partial(
        pl.kernel, mesh=_vmesh(),
        out_type=jax.ShapeDtypeStruct((B, D), F32),
        scratch_types=[pltpu.VMEM((M,), I32),
                       pltpu.VMEM((C,), I32),
                       pltpu.VMEM((C,), I32),
                       pltpu.VMEM((C, D), F32),
                       pltpu.SemaphoreType.DMA],
        compiler_params=_sc_cp(no_layout=True))
    def k(pos2_hbm, mcat_hbm, h2_hbm, out_hbm, pos2_v, midx_v, gidx_v,
          rows_v, sem):
        c = lax.axis_index("c")
        s = lax.axis_index("s")
        w = s * 2 + c
        pltpu.sync_copy(pos2_hbm, pos2_v)

        @pl.loop(0, nmax)
        def _(i):
            g = i * 32 + w

            @pl.when(g < nchunks)
            def _():
                base = g * C
                pltpu.sync_copy(mcat_hbm.at[pl.ds(base, C)], midx_v)
                for j in range(C // 16):
                    mv = midx_v[pl.ds(j * 16, 16)]
                    gidx_v[pl.ds(j * 16, 16)] = plsc.load_gather(pos2_v, [mv])
                pltpu.async_copy(h2_hbm.at[gidx_v], rows_v, sem).wait()
                pltpu.sync_copy(rows_v, out_hbm.at[pl.ds(base, C)])

    return k(pos2, mcat, h2)


def _tc_stats(h0):
    N, D = h0.shape
    BN = 2000

    def body(h_ref, o_ref):
        i = pl.program_id(0)
        blk = h_ref[...]
        row = jnp.concatenate([jnp.sum(blk, axis=0)[None, :],
                               jnp.sum(blk * blk, axis=0)[None, :],
                               jnp.zeros((1, 128 - 2 * D), F32)], axis=1)
        st = jnp.concatenate([row, jnp.zeros((7, 128), F32)], axis=0)

        @pl.when(i == 0)
        def _():
            o_ref[...] = st

        @pl.when(i > 0)
        def _():
            o_ref[...] += st

    return pl.pallas_call(
        body, grid=(N // BN,),
        in_specs=[pl.BlockSpec((BN, D), lambda i: (i, 0))],
        out_specs=pl.BlockSpec((8, 128), lambda i: (0, 0)),
        out_shape=jax.ShapeDtypeStruct((8, 128), F32))(h0)


def _tc_norm_matmul(h0, stats, gw, gb, gms, Wl, bl, Wr):
    N, D = h0.shape
    BN = 2000
    fN = float(N)

    def body(h_ref, st_ref, gw_ref, gb_ref, gms_ref, wl_ref, bl_ref, wr_ref,
             za_ref, zb_ref, r_ref):
        mean = st_ref[0:1, 0:D] / fN
        msq = st_ref[0:1, D:2 * D] / fN
        cvec = mean * gms_ref[...]
        var = msq - 2.0 * cvec * mean + cvec * cvec
        a = gw_ref[...] * lax.rsqrt(var + 1e-5)
        hn = (h_ref[...] - cvec) * a + gb_ref[...]
        z = jnp.dot(hn, wl_ref[...], precision=lax.Precision.HIGHEST,
                    preferred_element_type=F32)
        za_ref[...] = z[:, 0:16]
        zb_ref[...] = z[:, 16:32]
        r_ref[...] = jnp.dot(hn, wr_ref[...], precision=lax.Precision.HIGHEST,
                             preferred_element_type=F32) + bl_ref[...]

    full = lambda *shape: pl.BlockSpec(shape, lambda i: tuple(0 for _ in shape))
    return pl.pallas_call(
        body, grid=(N // BN,),
        in_specs=[pl.BlockSpec((BN, D), lambda i: (i, 0)),
                  full(8, 128), full(1, D), full(1, D), full(1, D),
                  full(D, D), full(1, D), full(D, D)],
        out_specs=[pl.BlockSpec((BN, 16), lambda i: (i, 0)),
                   pl.BlockSpec((BN, 16), lambda i: (i, 0)),
                   pl.BlockSpec((BN, D), lambda i: (i, 0))],
        out_shape=[jax.ShapeDtypeStruct((N, 16), F32),
                   jax.ShapeDtypeStruct((N, 16), F32),
                   jax.ShapeDtypeStruct((N, D), F32)])(
                       h0, stats, gw, gb, gms, Wl, bl, Wr)


def _tc_matmul2(hp, Wl, bl, Wr):
    N2, DD = hp.shape
    BN = 2000

    def body(h_ref, wl_ref, bl_ref, wr_ref, za_ref, zb_ref, r_ref):
        hp_blk = h_ref[...]
        z = jnp.dot(hp_blk, wl_ref[...], precision=lax.Precision.HIGHEST,
                    preferred_element_type=F32)
        za_ref[...] = z[:, 0:16]
        zb_ref[...] = z[:, 16:32]
        r_ref[...] = jnp.dot(hp_blk, wr_ref[...], precision=lax.Precision.HIGHEST,
                             preferred_element_type=F32) + bl_ref[...]

    full = lambda *shape: pl.BlockSpec(shape, lambda i: tuple(0 for _ in shape))
    return pl.pallas_call(
        body, grid=(N2 // BN,),
        in_specs=[pl.BlockSpec((BN, DD), lambda i: (i, 0)),
                  full(DD, 32), full(1, 32), full(DD, 32)],
        out_specs=[pl.BlockSpec((BN, 16), lambda i: (i, 0)),
                   pl.BlockSpec((BN, 16), lambda i: (i, 0)),
                   pl.BlockSpec((BN, 32), lambda i: (i, 0))],
        out_shape=[jax.ShapeDtypeStruct((N2, 16), F32),
                   jax.ShapeDtypeStruct((N2, 16), F32),
                   jax.ShapeDtypeStruct((N2, 32), F32)])(hp, Wl, bl, Wr)


def _tc_combine(aa, ab, cnt, r):
    N, D = r.shape
    BN = 2000

    def body(aa_ref, ab_ref, c_ref, r_ref, o_ref):
        inv = 1.0 / jnp.maximum(c_ref[:, 0:1], 1.0)
        agg = jnp.concatenate([aa_ref[...], ab_ref[...]], axis=1) * inv
        o_ref[...] = jnp.maximum(agg + r_ref[...], 0.0)

    return pl.pallas_call(
        body, grid=(N // BN,),
        in_specs=[pl.BlockSpec((BN, 16), lambda i: (i, 0)),
                  pl.BlockSpec((BN, 16), lambda i: (i, 0)),
                  pl.BlockSpec((BN, 16), lambda i: (i, 0)),
                  pl.BlockSpec((BN, D), lambda i: (i, 0))],
        out_specs=pl.BlockSpec((BN, D), lambda i: (i, 0)),
        out_shape=jax.ShapeDtypeStruct((N, D), F32))(aa, ab, cnt, r)


def _tc_project(g0, g1, wp0, wp1, bp):
    P, D = g0.shape

    def body(g0_ref, g1_ref, w0_ref, w1_ref, bp_ref, o_ref):
        acc = g0_ref[...] * w0_ref[...] + g1_ref[...] * w1_ref[...]
        o_ref[...] = jnp.sum(acc, axis=1, keepdims=True) + bp_ref[...]

    full = lambda *shape: pl.BlockSpec(shape, lambda: tuple(0 for _ in shape))
    return pl.pallas_call(
        body,
        in_specs=[full(P, D), full(P, D), full(1, D), full(1, D), full(1, 1)],
        out_specs=full(P, 1),
        out_shape=jax.ShapeDtypeStruct((P, 1), F32))(g0, g1, wp0, wp1, bp)


def kernel(x, edge1, edge2, pos1, pos2, mask0, mask1, emb_table,
           gn_weight, gn_bias, gn_mean_scale,
           Wl1, bl1, Wr1, Wl2, bl2, Wr2, Wp, bp):
    N, D = emb_table.shape
    N2 = pos1.shape[0]
    M = pos2.shape[0]
    P = mask0.shape[0]

    def pad_idx(a, T, pad_val):
        E = a.shape[0]
        Ep = -(-E // 1024) * 1024
        a = jnp.concatenate([a.astype(I32),
                             jnp.full((Ep - E,), pad_val, I32)])
        return a.reshape(-1, 128)

    x = x.astype(I32)
    src1_2d = pad_idx(edge1[0], N, 0)
    dst1_2d = pad_idx(edge1[1], N, N)
    src2_2d = pad_idx(edge2[0], N2, 0)
    dst2_2d = pad_idx(edge2[1], N2, N2)
    pos1f = pos1.astype(I32).reshape(-1)
    pos2 = pos2.astype(I32)
    mcat = jnp.concatenate([mask0, mask1]).astype(I32)

    gw = gn_weight.reshape(1, D)
    gb = gn_bias.reshape(1, D)
    gms = gn_mean_scale.reshape(1, D)
    bl1r = bl1.reshape(1, D)
    bl2r = bl2.reshape(1, 32)
    wp0 = Wp[:D].reshape(1, D)
    wp1 = Wp[D:].reshape(1, D)
    bpr = bp.reshape(1, 1)

    h0 = _sc_gather_rows(emb_table, x, D)
    stats = _tc_stats(h0)
    z1a, z1b, r1 = _tc_norm_matmul(h0, stats, gw, gb, gms, Wl1, bl1r, Wr1)

    a1, b1 = _sc_segment_sum(z1a, z1b, src1_2d, dst1_2d, N, bk=4)
    c1 = _sc_segment_count(dst1_2d, N)
    h1 = _tc_combine(a1, b1, c1, r1)

    h1p = _sc_gather_rows(h1, pos1f, D).reshape(N2, 2 * D)
    z2a, z2b, r2 = _tc_matmul2(h1p, Wl2, bl2r, Wr2)
    a2, b2, c2 = _sc_segment_sum(z2a, z2b, src2_2d, dst2_2d, N2, Rcnt=N2)
    h2 = _tc_combine(a2, b2, c2, r2)

    gcat = _sc_double_gather(pos2, mcat, h2)
    return _tc_project(gcat[:P], gcat[P:], wp0, wp1, bpr)


# speedup vs baseline: 8.7015x; 1.3658x over previous
import dataclasses
import functools

import jax
import jax.numpy as jnp
from jax import lax
from jax.experimental import pallas as pl
from jax.experimental.pallas import tpu as pltpu
from jax.experimental.pallas import tpu_sc as plsc

_MESH = dict(core_axis_name="c", subcore_axis_name="s")
C = 80
ZC = 40
F32 = jnp.float32
I32 = jnp.int32


def _vmesh():
    return plsc.VectorSubcoreMesh(**_MESH)


def _sc_cp(no_layout=False):
    cp = pltpu.CompilerParams()
    fields = pltpu.CompilerParams.__dataclass_fields__
    if "use_tc_tiling_on_sc" in fields:
        cp = dataclasses.replace(cp, use_tc_tiling_on_sc=False)
    if no_layout and "needs_layout_passes" in fields:
        cp = dataclasses.replace(cp, needs_layout_passes=False)
    return cp


def _fill(ref, n, value):
    @pl.loop(0, n)
    def _(i):
        ref.at[pl.ds(i, 1), pl.ds(0, 16)][...] = jnp.full((1, 16), value, F32)


def _sc_gather_rows(table, idx, D):
    B = idx.shape[0]
    nchunks = B // C
    nmax = (nchunks + 31) // 32

    @functools.partial(
        pl.kernel, mesh=_vmesh(),
        out_type=jax.ShapeDtypeStruct((B, D), F32),
        scratch_types=[pltpu.VMEM((C,), I32),
                       pltpu.VMEM((C, D), F32),
                       pltpu.SemaphoreType.DMA],
        compiler_params=_sc_cp())
    def k(table_hbm, idx_hbm, out_hbm, idx_v, rows_v, sem):
        c = lax.axis_index("c")
        s = lax.axis_index("s")
        w = s * 2 + c

        @pl.loop(0, nmax)
        def _(i):
            g = i * 32 + w

            @pl.when(g < nchunks)
            def _():
                base = g * C
                pltpu.sync_copy(idx_hbm.at[pl.ds(base, C)], idx_v)
                pltpu.async_copy(table_hbm.at[idx_v], rows_v, sem).wait()
                pltpu.sync_copy(rows_v, out_hbm.at[pl.ds(base, C)])

    return k(table, idx)


_BK = 8
_ZB = 125


def _zero_acc(acc_sh, zero_v, R, s):
    nchunks = R // _ZB
    nmax = (nchunks + 15) // 16

    @pl.loop(0, nmax)
    def _(i):
        g = i * 16 + s

        @pl.when(g < nchunks)
        def _():
            pltpu.sync_copy(zero_v, acc_sh.at[pl.ds(g * _ZB, _ZB)])


def _clamp_block(didx_bj, cidx_bj, lo, Rh, R):
    for t in range(128 // 16):
        v = didx_bj[pl.ds(t * 16, 16)]
        local = v - lo
        ok = (local >= 0) & (local < Rh)
        cidx_bj[pl.ds(t * 16, 16)] = jnp.where(ok, local, Rh)


def _sc_segment_sum(tab_a, tab_b, src2d, dst2d, R, Rcnt=None, bk=_BK):
    nrows = src2d.shape[0]
    nblk = nrows // bk
    nmax = (nblk + 15) // 16
    fused = Rcnt is not None
    Rh = (Rcnt // 2) if fused else 0
    out_t = [jax.ShapeDtypeStruct((R, 16), F32),
             jax.ShapeDtypeStruct((R, 16), F32)]
    scratch = [pltpu.VMEM((2, bk, 128), I32),
               pltpu.VMEM((2, bk, 128), I32),
               pltpu.VMEM((2, bk, 128, 16), F32),
               pltpu.VMEM((_ZB, 16), F32),
               pltpu.VMEM_SHARED((R + 8, 16), F32),
               pltpu.SemaphoreType.DMA,
               pltpu.SemaphoreType.DMA,
               pltpu.SemaphoreType.DMA]
    if fused:
        out_t.append(jax.ShapeDtypeStruct((Rcnt, 16), F32))
        scratch += [pltpu.VMEM((2, bk, 128), I32),
                    pltpu.VMEM((128, 16), F32),
                    pltpu.VMEM_SHARED((Rh + 8, 16), F32)]

    @functools.partial(pl.kernel, mesh=_vmesh(), out_type=tuple(out_t),
                       scratch_types=scratch, compiler_params=_sc_cp())
    def k(ta_hbm, tb_hbm, src_hbm, dst_hbm, oa_hbm, ob_hbm, *rest):
        if fused:
            (oc_hbm, sidx_v, didx_v, rows_v, zero_v, acc_sh,
             semI, semG, semS, cidx_v, ones_v, accc_sh) = rest
        else:
            (sidx_v, didx_v, rows_v, zero_v, acc_sh,
             semI, semG, semS) = rest
        c = lax.axis_index("c")
        s = lax.axis_index("s")
        nmine = (nblk + 15 - s) // 16
        _fill(zero_v, _ZB, 0.0)
        if fused:
            _fill(ones_v, 128, 1.0)
            lo = c * Rh
        _zero_acc(acc_sh, zero_v, R, s)
        if fused:
            _zero_acc(accc_sh, zero_v, Rh, s)
        plsc.subcore_barrier()

        tab = [ta_hbm, tb_hbm]
        nscat = bk * (2 if fused else 1)

        def fire_idx(b, i):
            r0 = (i * 16 + s) * bk
            pltpu.async_copy(src_hbm.at[pl.ds(r0, bk)], sidx_v.at[b], semI)
            pltpu.async_copy(dst_hbm.at[pl.ds(r0, bk)], didx_v.at[b], semI)

        def wait_idx(b):
            for _ in range(2):
                pltpu.make_async_copy(src_hbm.at[pl.ds(0, bk)],
                                      sidx_v.at[b], semI).wait()

        def drain_scatters():
            for _ in range(nscat):
                pltpu.make_async_copy(rows_v.at[0, 0],
                                      acc_sh.at[didx_v.at[0, 0]], semS).wait()

        @pl.when(nmine > 0)
        def _():
            fire_idx(0, 0)

        for half in range(2):
            @pl.when(c == half)
            def _():
                @pl.loop(0, (nmax + 1) // 2)
                def _(o):
                    for b in range(2):
                        i = o * 2 + b

                        @pl.when(i < nmine)
                        def _():
                            @pl.when(i >= 1)
                            def _():
                                drain_scatters()

                            @pl.when(i + 1 < nmine)
                            def _():
                                fire_idx(1 - b, i + 1)

                            wait_idx(b)
                            for j in range(bk):
                                pltpu.async_copy(
                                    tab[half].at[sidx_v.at[b, j]],
                                    rows_v.at[b, j], semG)
                            for j in range(bk):
                                pltpu.make_async_copy(
                                    tab[half].at[sidx_v.at[b, 0]],
                                    rows_v.at[b, 0], semG).wait()
                            if fused:
                                for j in range(bk):
                                    _clamp_block(didx_v.at[b, j],
                                                 cidx_v.at[b, j], lo, Rh, R)
                            for j in range(bk):
                                pltpu.async_copy(
                                    rows_v.at[b, j],
                                    acc_sh.at[didx_v.at[b, j]], semS,
                                    add=True)
                            if fused:
                                for j in range(bk):
                                    pltpu.async_copy(
                                        ones_v, accc_sh.at[cidx_v.at[b, j]],
                                        semS, add=True)

        @pl.when(nmine >= 1)
        def _():
            drain_scatters()

        plsc.subcore_barrier()
        R16 = R // 16
        for half in range(2):
            @pl.when(c == half)
            def _():
                pltpu.sync_copy(acc_sh.at[pl.ds(s * R16, R16)],
                                [oa_hbm, ob_hbm][half].at[pl.ds(s * R16, R16)])
        if fused:
            ncv = Rh // _ZB
            ncmax = (ncv + 15) // 16

            @pl.loop(0, ncmax)
            def _(i):
                g = i * 16 + s

                @pl.when(g < ncv)
                def _():
                    pltpu.sync_copy(accc_sh.at[pl.ds(g * _ZB, _ZB)],
                                    oc_hbm.at[pl.ds(lo + g * _ZB, _ZB)])

    return k(tab_a, tab_b, src2d, dst2d)


def _sc_segment_count(dst2d, R):
    nrows = dst2d.shape[0]
    nblk = nrows // _BK
    nmax = (nblk + 31) // 32

    @functools.partial(
        pl.kernel, mesh=_vmesh(),
        out_type=(jax.ShapeDtypeStruct((R, 16), F32),
                  jax.ShapeDtypeStruct((R, 16), F32)),
        scratch_types=[pltpu.VMEM((2, _BK, 128), I32),
                       pltpu.VMEM((2, _BK, 128), I32),
                       pltpu.VMEM((128, 16), F32),
                       pltpu.VMEM((_ZB, 16), F32),
                       pltpu.VMEM_SHARED((R + 8, 16), F32),
                       pltpu.SemaphoreType.DMA,
                       pltpu.SemaphoreType.DMA],
        compiler_params=_sc_cp())
    def k(dst_hbm, oa_hbm, ob_hbm, didx_v, cidx_v, ones_v, zero_v, acc_sh,
          semI, semS):
        c = lax.axis_index("c")
        s = lax.axis_index("s")
        w = s * 2 + c
        nmine = (nblk + 31 - w) // 32
        _fill(ones_v, 128, 1.0)
        _fill(zero_v, _ZB, 0.0)
        _zero_acc(acc_sh, zero_v, R, s)
        plsc.subcore_barrier()

        def fire_idx(b, i):
            r0 = (i * 32 + w) * _BK
            pltpu.async_copy(dst_hbm.at[pl.ds(r0, _BK)], didx_v.at[b], semI)

        @pl.when(nmine > 0)
        def _():
            fire_idx(0, 0)

        @pl.loop(0, (nmax + 1) // 2)
        def _(o):
            for b in range(2):
                i = o * 2 + b

                @pl.when(i < nmine)
                def _():
                    @pl.when(i >= 2)
                    def _():
                        for j in range(_BK):
                            pltpu.make_async_copy(
                                ones_v, acc_sh.at[cidx_v.at[0, 0]],
                                semS).wait()

                    @pl.when(i + 1 < nmine)
                    def _():
                        fire_idx(1 - b, i + 1)
                    pltpu.make_async_copy(dst_hbm.at[pl.ds(0, _BK)],
                                          didx_v.at[b], semI).wait()
                    for j in range(_BK):
                        _clamp_block(didx_v.at[b, j], cidx_v.at[b, j],
                                     0, R, R)
                    for j in range(_BK):
                        pltpu.async_copy(ones_v, acc_sh.at[cidx_v.at[b, j]],
                                         semS, add=True)

        for thresh in (1, 2):
            @pl.when(nmine >= thresh)
            def _():
                for j in range(_BK):
                    pltpu.make_async_copy(ones_v, acc_sh.at[cidx_v.at[0, 0]],
                                          semS).wait()

        plsc.subcore_barrier()
        R16 = R // 16
        for half in range(2):
            @pl.when(c == half)
            def _():
                pltpu.sync_copy(acc_sh.at[pl.ds(s * R16, R16)],
                                [oa_hbm, ob_hbm][half].at[pl.ds(s * R16, R16)])

    return k(dst2d)


def _sc_double_gather(pos2, mcat, h2):
    B = mcat.shape[0]
    M = pos2.shape[0]
    D = h2.shape[1]
    nchunks = B // C
    nmax = (nchunks + 31) // 32

    @functools.partial(
        pl.kernel, mesh=_vmesh(),
        out_type=jax.ShapeDtypeStruct((B, D), F32),
        scratch_types=[pltpu.VMEM((M,), I32),
                       pltpu.VMEM((C,), I32),
                       pltpu.VMEM((C,), I32),
                       pltpu.VMEM((C, D), F32),
                       pltpu.SemaphoreType.DMA],
        compiler_params=_sc_cp(no_layout=True))
    def k(pos2_hbm, mcat_hbm, h2_hbm, out_hbm, pos2_v, midx_v, gidx_v,
          rows_v, sem):
        c = lax.axis_index("c")
        s = lax.axis_index("s")
        w = s * 2 + c
        pltpu.sync_copy(pos2_hbm, pos2_v)

        @pl.loop(0, nmax)
        def _(i):
            g = i * 32 + w

            @pl.when(g < nchunks)
            def _():
                base = g * C
                pltpu.sync_copy(mcat_hbm.at[pl.ds(base, C)], midx_v)
                for j in range(C // 16):
                    mv = midx_v[pl.ds(j * 16, 16)]
                    gidx_v[pl.ds(j * 16, 16)] = plsc.load_gather(pos2_v, [mv])
                pltpu.async_copy(h2_hbm.at[gidx_v], rows_v, sem).wait()
                pltpu.sync_copy(rows_v, out_hbm.at[pl.ds(base, C)])

    return k(pos2, mcat, h2)


def _tc_stats(h0):
    N, D = h0.shape
    BN = 2000

    def body(h_ref, o_ref):
        i = pl.program_id(0)
        blk = h_ref[...]
        row = jnp.concatenate([jnp.sum(blk, axis=0)[None, :],
                               jnp.sum(blk * blk, axis=0)[None, :],
                               jnp.zeros((1, 128 - 2 * D), F32)], axis=1)
        st = jnp.concatenate([row, jnp.zeros((7, 128), F32)], axis=0)

        @pl.when(i == 0)
        def _():
            o_ref[...] = st

        @pl.when(i > 0)
        def _():
            o_ref[...] += st

    return pl.pallas_call(
        body, grid=(N // BN,),
        in_specs=[pl.BlockSpec((BN, D), lambda i: (i, 0))],
        out_specs=pl.BlockSpec((8, 128), lambda i: (0, 0)),
        out_shape=jax.ShapeDtypeStruct((8, 128), F32))(h0)


def _tc_norm_matmul(h0, stats, gw, gb, gms, Wl, bl, Wr):
    N, D = h0.shape
    BN = 2000
    fN = float(N)

    def body(h_ref, st_ref, gw_ref, gb_ref, gms_ref, wl_ref, bl_ref, wr_ref,
             za_ref, zb_ref, r_ref):
        mean = st_ref[0:1, 0:D] / fN
        msq = st_ref[0:1, D:2 * D] / fN
        cvec = mean * gms_ref[...]
        var = msq - 2.0 * cvec * mean + cvec * cvec
        a = gw_ref[...] * lax.rsqrt(var + 1e-5)
        hn = (h_ref[...] - cvec) * a + gb_ref[...]
        z = jnp.dot(hn, wl_ref[...], precision=lax.Precision.HIGHEST,
                    preferred_element_type=F32)
        za_ref[...] = z[:, 0:16]
        zb_ref[...] = z[:, 16:32]
        r_ref[...] = jnp.dot(hn, wr_ref[...], precision=lax.Precision.HIGHEST,
                             preferred_element_type=F32) + bl_ref[...]

    full = lambda *shape: pl.BlockSpec(shape, lambda i: tuple(0 for _ in shape))
    return pl.pallas_call(
        body, grid=(N // BN,),
        in_specs=[pl.BlockSpec((BN, D), lambda i: (i, 0)),
                  full(8, 128), full(1, D), full(1, D), full(1, D),
                  full(D, D), full(1, D), full(D, D)],
        out_specs=[pl.BlockSpec((BN, 16), lambda i: (i, 0)),
                   pl.BlockSpec((BN, 16), lambda i: (i, 0)),
                   pl.BlockSpec((BN, D), lambda i: (i, 0))],
        out_shape=[jax.ShapeDtypeStruct((N, 16), F32),
                   jax.ShapeDtypeStruct((N, 16), F32),
                   jax.ShapeDtypeStruct((N, D), F32)])(
                       h0, stats, gw, gb, gms, Wl, bl, Wr)


def _tc_matmul2(hp, Wl, bl, Wr):
    N2, DD = hp.shape
    BN = 2000

    def body(h_ref, wl_ref, bl_ref, wr_ref, za_ref, zb_ref, r_ref):
        hp_blk = h_ref[...]
        z = jnp.dot(hp_blk, wl_ref[...], precision=lax.Precision.HIGHEST,
                    preferred_element_type=F32)
        za_ref[...] = z[:, 0:16]
        zb_ref[...] = z[:, 16:32]
        r_ref[...] = jnp.dot(hp_blk, wr_ref[...], precision=lax.Precision.HIGHEST,
                             preferred_element_type=F32) + bl_ref[...]

    full = lambda *shape: pl.BlockSpec(shape, lambda i: tuple(0 for _ in shape))
    return pl.pallas_call(
        body, grid=(N2 // BN,),
        in_specs=[pl.BlockSpec((BN, DD), lambda i: (i, 0)),
                  full(DD, 32), full(1, 32), full(DD, 32)],
        out_specs=[pl.BlockSpec((BN, 16), lambda i: (i, 0)),
                   pl.BlockSpec((BN, 16), lambda i: (i, 0)),
                   pl.BlockSpec((BN, 32), lambda i: (i, 0))],
        out_shape=[jax.ShapeDtypeStruct((N2, 16), F32),
                   jax.ShapeDtypeStruct((N2, 16), F32),
                   jax.ShapeDtypeStruct((N2, 32), F32)])(hp, Wl, bl, Wr)


def _tc_combine2(aa, ab, ca, cb, r):
    N, D = r.shape
    BN = 2000

    def body(aa_ref, ab_ref, ca_ref, cb_ref, r_ref, o_ref):
        cnt = ca_ref[:, 0:1] + cb_ref[:, 0:1]
        inv = 1.0 / jnp.maximum(cnt, 1.0)
        agg = jnp.concatenate([aa_ref[...], ab_ref[...]], axis=1) * inv
        o_ref[...] = jnp.maximum(agg + r_ref[...], 0.0)

    return pl.pallas_call(
        body, grid=(N // BN,),
        in_specs=[pl.BlockSpec((BN, 16), lambda i: (i, 0)),
                  pl.BlockSpec((BN, 16), lambda i: (i, 0)),
                  pl.BlockSpec((BN, 16), lambda i: (i, 0)),
                  pl.BlockSpec((BN, 16), lambda i: (i, 0)),
                  pl.BlockSpec((BN, D), lambda i: (i, 0))],
        out_specs=pl.BlockSpec((BN, D), lambda i: (i, 0)),
        out_shape=jax.ShapeDtypeStruct((N, D), F32))(aa, ab, ca, cb, r)


def _tc_combine(aa, ab, cnt, r):
    N, D = r.shape
    BN = 2000

    def body(aa_ref, ab_ref, c_ref, r_ref, o_ref):
        inv = 1.0 / jnp.maximum(c_ref[:, 0:1], 1.0)
        agg = jnp.concatenate([aa_ref[...], ab_ref[...]], axis=1) * inv
        o_ref[...] = jnp.maximum(agg + r_ref[...], 0.0)

    return pl.pallas_call(
        body, grid=(N // BN,),
        in_specs=[pl.BlockSpec((BN, 16), lambda i: (i, 0)),
                  pl.BlockSpec((BN, 16), lambda i: (i, 0)),
                  pl.BlockSpec((BN, 16), lambda i: (i, 0)),
                  pl.BlockSpec((BN, D), lambda i: (i, 0))],
        out_specs=pl.BlockSpec((BN, D), lambda i: (i, 0)),
        out_shape=jax.ShapeDtypeStruct((N, D), F32))(aa, ab, cnt, r)


def _tc_project(g0, g1, wp0, wp1, bp):
    P, D = g0.shape

    def body(g0_ref, g1_ref, w0_ref, w1_ref, bp_ref, o_ref):
        acc = g0_ref[...] * w0_ref[...] + g1_ref[...] * w1_ref[...]
        o_ref[...] = jnp.sum(acc, axis=1, keepdims=True) + bp_ref[...]

    full = lambda *shape: pl.BlockSpec(shape, lambda: tuple(0 for _ in shape))
    return pl.pallas_call(
        body,
        in_specs=[full(P, D), full(P, D), full(1, D), full(1, D), full(1, 1)],
        out_specs=full(P, 1),
        out_shape=jax.ShapeDtypeStruct((P, 1), F32))(g0, g1, wp0, wp1, bp)


def kernel(x, edge1, edge2, pos1, pos2, mask0, mask1, emb_table,
           gn_weight, gn_bias, gn_mean_scale,
           Wl1, bl1, Wr1, Wl2, bl2, Wr2, Wp, bp):
    N, D = emb_table.shape
    N2 = pos1.shape[0]
    M = pos2.shape[0]
    P = mask0.shape[0]

    def pad_idx(a, T, pad_val):
        E = a.shape[0]
        Ep = -(-E // 1024) * 1024
        a = jnp.concatenate([a.astype(I32),
                             jnp.full((Ep - E,), pad_val, I32)])
        return a.reshape(-1, 128)

    x = x.astype(I32)
    src1_2d = pad_idx(edge1[0], N, 0)
    dst1_2d = pad_idx(edge1[1], N, N)
    src2_2d = pad_idx(edge2[0], N2, 0)
    dst2_2d = pad_idx(edge2[1], N2, N2)
    pos1f = pos1.astype(I32).reshape(-1)
    pos2 = pos2.astype(I32)
    mcat = jnp.concatenate([mask0, mask1]).astype(I32)

    gw = gn_weight.reshape(1, D)
    gb = gn_bias.reshape(1, D)
    gms = gn_mean_scale.reshape(1, D)
    bl1r = bl1.reshape(1, D)
    bl2r = bl2.reshape(1, 32)
    wp0 = Wp[:D].reshape(1, D)
    wp1 = Wp[D:].reshape(1, D)
    bpr = bp.reshape(1, 1)

    h0 = _sc_gather_rows(emb_table, x, D)
    c1a, c1b = _sc_segment_count(dst1_2d, N)
    stats = _tc_stats(h0)
    z1a, z1b, r1 = _tc_norm_matmul(h0, stats, gw, gb, gms, Wl1, bl1r, Wr1)

    a1, b1 = _sc_segment_sum(z1a, z1b, src1_2d, dst1_2d, N, bk=4)
    h1 = _tc_combine2(a1, b1, c1a, c1b, r1)

    h1p = _sc_gather_rows(h1, pos1f, D).reshape(N2, 2 * D)
    z2a, z2b, r2 = _tc_matmul2(h1p, Wl2, bl2r, Wr2)
    a2, b2, c2 = _sc_segment_sum(z2a, z2b, src2_2d, dst2_2d, N2, Rcnt=N2)
    h2 = _tc_combine(a2, b2, c2, r2)

    gcat = _sc_double_gather(pos2, mcat, h2)
    return _tc_project(gcat[:P], gcat[P:], wp0, wp1, bpr)


# speedup vs baseline: 10.9323x; 1.2564x over previous
import dataclasses
import functools

import jax
import jax.numpy as jnp
from jax import lax
from jax.experimental import pallas as pl
from jax.experimental.pallas import tpu as pltpu
from jax.experimental.pallas import tpu_sc as plsc

_MESH = dict(core_axis_name="c", subcore_axis_name="s")
C = 80
ZC = 40
F32 = jnp.float32
I32 = jnp.int32


def _vmesh():
    return plsc.VectorSubcoreMesh(**_MESH)


def _sc_cp(no_layout=False):
    cp = pltpu.CompilerParams()
    fields = pltpu.CompilerParams.__dataclass_fields__
    if "use_tc_tiling_on_sc" in fields:
        cp = dataclasses.replace(cp, use_tc_tiling_on_sc=False)
    if no_layout and "needs_layout_passes" in fields:
        cp = dataclasses.replace(cp, needs_layout_passes=False)
    return cp


def _fill(ref, n, value):
    @pl.loop(0, n)
    def _(i):
        ref.at[pl.ds(i, 1), pl.ds(0, 16)][...] = jnp.full((1, 16), value, F32)


def _sc_gather_rows(table, idx, D):
    B = idx.shape[0]
    nchunks = B // C
    nmax = (nchunks + 31) // 32

    @functools.partial(
        pl.kernel, mesh=_vmesh(),
        out_type=jax.ShapeDtypeStruct((B, D), F32),
        scratch_types=[pltpu.VMEM((C,), I32),
                       pltpu.VMEM((C, D), F32),
                       pltpu.SemaphoreType.DMA],
        compiler_params=_sc_cp())
    def k(table_hbm, idx_hbm, out_hbm, idx_v, rows_v, sem):
        c = lax.axis_index("c")
        s = lax.axis_index("s")
        w = s * 2 + c

        @pl.loop(0, nmax)
        def _(i):
            g = i * 32 + w

            @pl.when(g < nchunks)
            def _():
                base = g * C
                pltpu.sync_copy(idx_hbm.at[pl.ds(base, C)], idx_v)
                pltpu.async_copy(table_hbm.at[idx_v], rows_v, sem).wait()
                pltpu.sync_copy(rows_v, out_hbm.at[pl.ds(base, C)])

    return k(table, idx)


_BK = 8
_ZB = 125


def _zero_acc(acc_sh, zero_v, R, s):
    nchunks = R // _ZB
    nmax = (nchunks + 15) // 16

    @pl.loop(0, nmax)
    def _(i):
        g = i * 16 + s

        @pl.when(g < nchunks)
        def _():
            pltpu.sync_copy(zero_v, acc_sh.at[pl.ds(g * _ZB, _ZB)])


def _clamp_block(didx_bj, cidx_bj, lo, Rh, R):
    for t in range(128 // 16):
        v = didx_bj[pl.ds(t * 16, 16)]
        local = v - lo
        ok = (local >= 0) & (local < Rh)
        cidx_bj[pl.ds(t * 16, 16)] = jnp.where(ok, local, Rh)


def _sc_segment_sum(tab_a, tab_b, src2d, dst2d, R, Rcnt=None, bk=_BK):
    nrows = src2d.shape[0]
    nblk = nrows // bk
    nmax = (nblk + 15) // 16
    fused = Rcnt is not None
    Rh = (Rcnt // 2) if fused else 0
    out_t = [jax.ShapeDtypeStruct((R, 16), F32),
             jax.ShapeDtypeStruct((R, 16), F32)]
    scratch = [pltpu.VMEM((2, bk, 128), I32),
               pltpu.VMEM((2, bk, 128), I32),
               pltpu.VMEM((2, bk, 128, 16), F32),
               pltpu.VMEM((_ZB, 16), F32),
               pltpu.VMEM_SHARED((R + 8, 16), F32),
               pltpu.SemaphoreType.DMA,
               pltpu.SemaphoreType.DMA,
               pltpu.SemaphoreType.DMA]
    if fused:
        out_t.append(jax.ShapeDtypeStruct((Rcnt, 16), F32))
        scratch += [pltpu.VMEM((2, bk, 128), I32),
                    pltpu.VMEM((128, 16), F32),
                    pltpu.VMEM_SHARED((Rh + 8, 16), F32)]

    @functools.partial(pl.kernel, mesh=_vmesh(), out_type=tuple(out_t),
                       scratch_types=scratch, compiler_params=_sc_cp())
    def k(ta_hbm, tb_hbm, src_hbm, dst_hbm, oa_hbm, ob_hbm, *rest):
        if fused:
            (oc_hbm, sidx_v, didx_v, rows_v, zero_v, acc_sh,
             semI, semG, semS, cidx_v, ones_v, accc_sh) = rest
        else:
            (sidx_v, didx_v, rows_v, zero_v, acc_sh,
             semI, semG, semS) = rest
        c = lax.axis_index("c")
        s = lax.axis_index("s")
        nmine = (nblk + 15 - s) // 16
        _fill(zero_v, _ZB, 0.0)
        if fused:
            _fill(ones_v, 128, 1.0)
            lo = c * Rh
        _zero_acc(acc_sh, zero_v, R, s)
        if fused:
            _zero_acc(accc_sh, zero_v, Rh, s)
        plsc.subcore_barrier()

        tab = [ta_hbm, tb_hbm]
        nscat = bk * (2 if fused else 1)

        def fire_idx(b, i):
            r0 = (i * 16 + s) * bk
            pltpu.async_copy(src_hbm.at[pl.ds(r0, bk)], sidx_v.at[b], semI)
            pltpu.async_copy(dst_hbm.at[pl.ds(r0, bk)], didx_v.at[b], semI)

        def wait_idx(b):
            for _ in range(2):
                pltpu.make_async_copy(src_hbm.at[pl.ds(0, bk)],
                                      sidx_v.at[b], semI).wait()

        def drain_scatters():
            for _ in range(nscat):
                pltpu.make_async_copy(rows_v.at[0, 0],
                                      acc_sh.at[didx_v.at[0, 0]], semS).wait()

        @pl.when(nmine > 0)
        def _():
            fire_idx(0, 0)

        for half in range(2):
            @pl.when(c == half)
            def _():
                @pl.loop(0, (nmax + 1) // 2)
                def _(o):
                    for b in range(2):
                        i = o * 2 + b

                        @pl.when(i < nmine)
                        def _():
                            @pl.when(i >= 1)
                            def _():
                                drain_scatters()

                            @pl.when(i + 1 < nmine)
                            def _():
                                fire_idx(1 - b, i + 1)

                            wait_idx(b)
                            for j in range(bk):
                                pltpu.async_copy(
                                    tab[half].at[sidx_v.at[b, j]],
                                    rows_v.at[b, j], semG)
                            for j in range(bk):
                                pltpu.make_async_copy(
                                    tab[half].at[sidx_v.at[b, 0]],
                                    rows_v.at[b, 0], semG).wait()
                            if fused:
                                for j in range(bk):
                                    _clamp_block(didx_v.at[b, j],
                                                 cidx_v.at[b, j], lo, Rh, R)
                            for j in range(bk):
                                pltpu.async_copy(
                                    rows_v.at[b, j],
                                    acc_sh.at[didx_v.at[b, j]], semS,
                                    add=True)
                            if fused:
                                for j in range(bk):
                                    pltpu.async_copy(
                                        ones_v, accc_sh.at[cidx_v.at[b, j]],
                                        semS, add=True)

        @pl.when(nmine >= 1)
        def _():
            drain_scatters()

        plsc.subcore_barrier()
        R16 = R // 16
        for half in range(2):
            @pl.when(c == half)
            def _():
                pltpu.sync_copy(acc_sh.at[pl.ds(s * R16, R16)],
                                [oa_hbm, ob_hbm][half].at[pl.ds(s * R16, R16)])
        if fused:
            ncv = Rh // _ZB
            ncmax = (ncv + 15) // 16

            @pl.loop(0, ncmax)
            def _(i):
                g = i * 16 + s

                @pl.when(g < ncv)
                def _():
                    pltpu.sync_copy(accc_sh.at[pl.ds(g * _ZB, _ZB)],
                                    oc_hbm.at[pl.ds(lo + g * _ZB, _ZB)])

    return k(tab_a, tab_b, src2d, dst2d)


def _sc_segment_count(dst2d, R):
    nrows = dst2d.shape[0]
    nblk = nrows // _BK
    nmax = (nblk + 31) // 32

    @functools.partial(
        pl.kernel, mesh=_vmesh(),
        out_type=(jax.ShapeDtypeStruct((R, 16), F32),
                  jax.ShapeDtypeStruct((R, 16), F32)),
        scratch_types=[pltpu.VMEM((2, _BK, 128), I32),
                       pltpu.VMEM((2, _BK, 128), I32),
                       pltpu.VMEM((128, 16), F32),
                       pltpu.VMEM((_ZB, 16), F32),
                       pltpu.VMEM_SHARED((R + 8, 16), F32),
                       pltpu.SemaphoreType.DMA,
                       pltpu.SemaphoreType.DMA],
        compiler_params=_sc_cp())
    def k(dst_hbm, oa_hbm, ob_hbm, didx_v, cidx_v, ones_v, zero_v, acc_sh,
          semI, semS):
        c = lax.axis_index("c")
        s = lax.axis_index("s")
        w = s * 2 + c
        nmine = (nblk + 31 - w) // 32
        _fill(ones_v, 128, 1.0)
        _fill(zero_v, _ZB, 0.0)
        _zero_acc(acc_sh, zero_v, R, s)
        plsc.subcore_barrier()

        def fire_idx(b, i):
            r0 = (i * 32 + w) * _BK
            pltpu.async_copy(dst_hbm.at[pl.ds(r0, _BK)], didx_v.at[b], semI)

        @pl.when(nmine > 0)
        def _():
            fire_idx(0, 0)

        @pl.loop(0, (nmax + 1) // 2)
        def _(o):
            for b in range(2):
                i = o * 2 + b

                @pl.when(i < nmine)
                def _():
                    @pl.when(i >= 2)
                    def _():
                        for j in range(_BK):
                            pltpu.make_async_copy(
                                ones_v, acc_sh.at[cidx_v.at[0, 0]],
                                semS).wait()

                    @pl.when(i + 1 < nmine)
                    def _():
                        fire_idx(1 - b, i + 1)
                    pltpu.make_async_copy(dst_hbm.at[pl.ds(0, _BK)],
                                          didx_v.at[b], semI).wait()
                    for j in range(_BK):
                        _clamp_block(didx_v.at[b, j], cidx_v.at[b, j],
                                     0, R, R)
                    for j in range(_BK):
                        pltpu.async_copy(ones_v, acc_sh.at[cidx_v.at[b, j]],
                                         semS, add=True)

        for thresh in (1, 2):
            @pl.when(nmine >= thresh)
            def _():
                for j in range(_BK):
                    pltpu.make_async_copy(ones_v, acc_sh.at[cidx_v.at[0, 0]],
                                          semS).wait()

        plsc.subcore_barrier()
        R16 = R // 16
        for half in range(2):
            @pl.when(c == half)
            def _():
                pltpu.sync_copy(acc_sh.at[pl.ds(s * R16, R16)],
                                [oa_hbm, ob_hbm][half].at[pl.ds(s * R16, R16)])

    return k(dst2d)


def _sc_double_gather(pos2, mcat, h2):
    B = mcat.shape[0]
    M = pos2.shape[0]
    D = h2.shape[1]
    nchunks = B // C
    nmax = (nchunks + 31) // 32

    @functools.partial(
        pl.kernel, mesh=_vmesh(),
        out_type=jax.ShapeDtypeStruct((B, D), F32),
        scratch_types=[pltpu.VMEM((M,), I32),
                       pltpu.VMEM((C,), I32),
                       pltpu.VMEM((C,), I32),
                       pltpu.VMEM((C, D), F32),
                       pltpu.SemaphoreType.DMA],
        compiler_params=_sc_cp(no_layout=True))
    def k(pos2_hbm, mcat_hbm, h2_hbm, out_hbm, pos2_v, midx_v, gidx_v,
          rows_v, sem):
        c = lax.axis_index("c")
        s = lax.axis_index("s")
        w = s * 2 + c
        pltpu.sync_copy(pos2_hbm, pos2_v)

        @pl.loop(0, nmax)
        def _(i):
            g = i * 32 + w

            @pl.when(g < nchunks)
            def _():
                base = g * C
                pltpu.sync_copy(mcat_hbm.at[pl.ds(base, C)], midx_v)
                for j in range(C // 16):
                    mv = midx_v[pl.ds(j * 16, 16)]
                    gidx_v[pl.ds(j * 16, 16)] = plsc.load_gather(pos2_v, [mv])
                pltpu.async_copy(h2_hbm.at[gidx_v], rows_v, sem).wait()
                pltpu.sync_copy(rows_v, out_hbm.at[pl.ds(base, C)])

    return k(pos2, mcat, h2)


def _tc_stats(h0):
    N, D = h0.shape
    BN = 2000

    def body(h_ref, o_ref):
        i = pl.program_id(0)
        blk = h_ref[...]
        row = jnp.concatenate([jnp.sum(blk, axis=0)[None, :],
                               jnp.sum(blk * blk, axis=0)[None, :],
                               jnp.zeros((1, 128 - 2 * D), F32)], axis=1)
        st = jnp.concatenate([row, jnp.zeros((7, 128), F32)], axis=0)

        @pl.when(i == 0)
        def _():
            o_ref[...] = st

        @pl.when(i > 0)
        def _():
            o_ref[...] += st

    return pl.pallas_call(
        body, grid=(N // BN,),
        in_specs=[pl.BlockSpec((BN, D), lambda i: (i, 0))],
        out_specs=pl.BlockSpec((8, 128), lambda i: (0, 0)),
        out_shape=jax.ShapeDtypeStruct((8, 128), F32))(h0)


def _tc_norm_matmul(h0, stats, gw, gb, gms, Wl, bl, Wr):
    N, D = h0.shape
    BN = 2000
    fN = float(N)

    def body(h_ref, st_ref, gw_ref, gb_ref, gms_ref, wl_ref, bl_ref, wr_ref,
             za_ref, zb_ref, r_ref):
        mean = st_ref[0:1, 0:D] / fN
        msq = st_ref[0:1, D:2 * D] / fN
        cvec = mean * gms_ref[...]
        var = msq - 2.0 * cvec * mean + cvec * cvec
        a = gw_ref[...] * lax.rsqrt(var + 1e-5)
        hn = (h_ref[...] - cvec) * a + gb_ref[...]
        z = jnp.dot(hn, wl_ref[...], precision=lax.Precision.HIGHEST,
                    preferred_element_type=F32)
        za_ref[...] = z[:, 0:16]
        zb_ref[...] = z[:, 16:32]
        r_ref[...] = jnp.dot(hn, wr_ref[...], precision=lax.Precision.HIGHEST,
                             preferred_element_type=F32) + bl_ref[...]

    full = lambda *shape: pl.BlockSpec(shape, lambda i: tuple(0 for _ in shape))
    return pl.pallas_call(
        body, grid=(N // BN,),
        in_specs=[pl.BlockSpec((BN, D), lambda i: (i, 0)),
                  full(8, 128), full(1, D), full(1, D), full(1, D),
                  full(D, D), full(1, D), full(D, D)],
        out_specs=[pl.BlockSpec((BN, 16), lambda i: (i, 0)),
                   pl.BlockSpec((BN, 16), lambda i: (i, 0)),
                   pl.BlockSpec((BN, D), lambda i: (i, 0))],
        out_shape=[jax.ShapeDtypeStruct((N, 16), F32),
                   jax.ShapeDtypeStruct((N, 16), F32),
                   jax.ShapeDtypeStruct((N, D), F32)])(
                       h0, stats, gw, gb, gms, Wl, bl, Wr)


def _tc_matmul2(hp, Wl, bl, Wr):
    N2, DD = hp.shape
    BN = 2000

    def body(h_ref, wl_ref, bl_ref, wr_ref, za_ref, zb_ref, r_ref):
        hp_blk = h_ref[...]
        z = jnp.dot(hp_blk, wl_ref[...], precision=lax.Precision.HIGHEST,
                    preferred_element_type=F32)
        za_ref[...] = z[:, 0:16]
        zb_ref[...] = z[:, 16:32]
        r_ref[...] = jnp.dot(hp_blk, wr_ref[...], precision=lax.Precision.HIGHEST,
                             preferred_element_type=F32) + bl_ref[...]

    full = lambda *shape: pl.BlockSpec(shape, lambda i: tuple(0 for _ in shape))
    return pl.pallas_call(
        body, grid=(N2 // BN,),
        in_specs=[pl.BlockSpec((BN, DD), lambda i: (i, 0)),
                  full(DD, 32), full(1, 32), full(DD, 32)],
        out_specs=[pl.BlockSpec((BN, 16), lambda i: (i, 0)),
                   pl.BlockSpec((BN, 16), lambda i: (i, 0)),
                   pl.BlockSpec((BN, 32), lambda i: (i, 0))],
        out_shape=[jax.ShapeDtypeStruct((N2, 16), F32),
                   jax.ShapeDtypeStruct((N2, 16), F32),
                   jax.ShapeDtypeStruct((N2, 32), F32)])(hp, Wl, bl, Wr)


def _tc_combine2(aa, ab, ca, cb, r):
    N, D = r.shape
    BN = 2000

    def body(aa_ref, ab_ref, ca_ref, cb_ref, r_ref, o_ref):
        cnt = ca_ref[:, 0:1] + cb_ref[:, 0:1]
        inv = 1.0 / jnp.maximum(cnt, 1.0)
        agg = jnp.concatenate([aa_ref[...], ab_ref[...]], axis=1) * inv
        o_ref[...] = jnp.maximum(agg + r_ref[...], 0.0)

    return pl.pallas_call(
        body, grid=(N // BN,),
        in_specs=[pl.BlockSpec((BN, 16), lambda i: (i, 0)),
                  pl.BlockSpec((BN, 16), lambda i: (i, 0)),
                  pl.BlockSpec((BN, 16), lambda i: (i, 0)),
                  pl.BlockSpec((BN, 16), lambda i: (i, 0)),
                  pl.BlockSpec((BN, D), lambda i: (i, 0))],
        out_specs=pl.BlockSpec((BN, D), lambda i: (i, 0)),
        out_shape=jax.ShapeDtypeStruct((N, D), F32))(aa, ab, ca, cb, r)


def _tc_combine(aa, ab, cnt, r):
    N, D = r.shape
    BN = 2000

    def body(aa_ref, ab_ref, c_ref, r_ref, o_ref):
        inv = 1.0 / jnp.maximum(c_ref[:, 0:1], 1.0)
        agg = jnp.concatenate([aa_ref[...], ab_ref[...]], axis=1) * inv
        o_ref[...] = jnp.maximum(agg + r_ref[...], 0.0)

    return pl.pallas_call(
        body, grid=(N // BN,),
        in_specs=[pl.BlockSpec((BN, 16), lambda i: (i, 0)),
                  pl.BlockSpec((BN, 16), lambda i: (i, 0)),
                  pl.BlockSpec((BN, 16), lambda i: (i, 0)),
                  pl.BlockSpec((BN, D), lambda i: (i, 0))],
        out_specs=pl.BlockSpec((BN, D), lambda i: (i, 0)),
        out_shape=jax.ShapeDtypeStruct((N, D), F32))(aa, ab, cnt, r)


def _tc_project(g0, g1, wp0, wp1, bp):
    P, D = g0.shape

    def body(g0_ref, g1_ref, w0_ref, w1_ref, bp_ref, o_ref):
        acc = g0_ref[...] * w0_ref[...] + g1_ref[...] * w1_ref[...]
        o_ref[...] = jnp.sum(acc, axis=1, keepdims=True) + bp_ref[...]

    full = lambda *shape: pl.BlockSpec(shape, lambda: tuple(0 for _ in shape))
    return pl.pallas_call(
        body,
        in_specs=[full(P, D), full(P, D), full(1, D), full(1, D), full(1, 1)],
        out_specs=full(P, 1),
        out_shape=jax.ShapeDtypeStruct((P, 1), F32))(g0, g1, wp0, wp1, bp)


def kernel(x, edge1, edge2, pos1, pos2, mask0, mask1, emb_table,
           gn_weight, gn_bias, gn_mean_scale,
           Wl1, bl1, Wr1, Wl2, bl2, Wr2, Wp, bp):
    N, D = emb_table.shape
    N2 = pos1.shape[0]
    M = pos2.shape[0]
    P = mask0.shape[0]

    def pad_idx(a, T, pad_val):
        E = a.shape[0]
        Ep = -(-E // 1024) * 1024
        a = jnp.concatenate([a.astype(I32),
                             jnp.full((Ep - E,), pad_val, I32)])
        return a.reshape(-1, 128)

    x = x.astype(I32)
    src1_2d = pad_idx(edge1[0], N, 0)
    dst1_2d = pad_idx(edge1[1], N, N)
    src2_2d = pad_idx(edge2[0], N2, 0)
    dst2_2d = pad_idx(edge2[1], N2, N2)
    pos1f = pos1.astype(I32).reshape(-1)
    pos2 = pos2.astype(I32)
    mcat = jnp.concatenate([mask0, mask1]).astype(I32)

    gw = gn_weight.reshape(1, D)
    gb = gn_bias.reshape(1, D)
    gms = gn_mean_scale.reshape(1, D)
    bl1r = bl1.reshape(1, D)
    bl2r = bl2.reshape(1, 32)
    wp0 = Wp[:D].reshape(1, D)
    wp1 = Wp[D:].reshape(1, D)
    bpr = bp.reshape(1, 1)

    h0 = _sc_gather_rows(emb_table, x, D)
    c1a, c1b = _sc_segment_count(dst1_2d, N)
    c2a, c2b = _sc_segment_count(dst2_2d, N2)
    stats = _tc_stats(h0)
    z1a, z1b, r1 = _tc_norm_matmul(h0, stats, gw, gb, gms, Wl1, bl1r, Wr1)

    a1, b1 = _sc_segment_sum(z1a, z1b, src1_2d, dst1_2d, N, bk=4)
    h1 = _tc_combine2(a1, b1, c1a, c1b, r1)

    h1p = _sc_gather_rows(h1, pos1f, D).reshape(N2, 2 * D)
    z2a, z2b, r2 = _tc_matmul2(h1p, Wl2, bl2r, Wr2)
    a2, b2 = _sc_segment_sum(z2a, z2b, src2_2d, dst2_2d, N2)
    h2 = _tc_combine2(a2, b2, c2a, c2b, r2)

    gcat = _sc_double_gather(pos2, mcat, h2)
    return _tc_project(gcat[:P], gcat[P:], wp0, wp1, bpr)
